# baseline (speedup 1.0000x reference)
import jax
import jax.numpy as jnp
from jax.experimental import pallas as pl

G = 7


def kernel(x):
    sorted_indices = jnp.argsort(-x, axis=-1)
    sorted_x = jnp.take_along_axis(x, sorted_indices, axis=-1)
    winner = jax.nn.softmax(sorted_x[:, :G], axis=-1)
    loser = -jax.nn.softmax(1.0 - sorted_x[:, -G:], axis=-1)
    zeros = jnp.zeros_like(sorted_x[:, G:-G])
    b_c = jnp.concatenate([winner, zeros, loser], axis=1)
    return (b_c, sorted_indices)


# speedup vs baseline: 1.1661x; 1.1661x over previous
import functools

import jax
import jax.numpy as jnp
from jax import lax
from jax.experimental import pallas as pl
from jax.experimental.pallas import tpu as pltpu
from jax.experimental.pallas import tpu_sc as plsc

_G = 7
_B, _N = 128, 32768
_L = 16
_SEG = _N // _L
_RADIX = 256
_NC, _NS = 2, 16
_NW = _NC * _NS
_RPW = _B // _NW
_MSB = -2147483648


def _vec_iota():
    return lax.iota(jnp.int32, _L)


def _digit(k_i32, shift):
    return jnp.bitwise_and(jnp.right_shift(k_i32, shift), 255)


@functools.partial(
    pl.kernel,
    mesh=plsc.VectorSubcoreMesh(core_axis_name="c", subcore_axis_name="s"),
    compiler_params=pltpu.CompilerParams(needs_layout_passes=False),
    out_type=(
        jax.ShapeDtypeStruct((_B * _N,), jnp.float32),
        jax.ShapeDtypeStruct((_B * _N,), jnp.int32),
    ),
    scratch_types=[
        pltpu.VMEM((_N,), jnp.float32),
        pltpu.VMEM((_N,), jnp.int32),
        pltpu.VMEM((_N,), jnp.int32),
        pltpu.VMEM((_RADIX * _L,), jnp.int32),
        pltpu.VMEM((4096,), jnp.float32),
        pltpu.VMEM((_L,), jnp.float32),
        pltpu.VMEM((_L,), jnp.float32),
    ],
)
def _sc_sort(x_hbm, bc_hbm, idx_hbm, kv, idx_a, idx_b, hist2, zbuf, headv, tailv):
    wid = lax.axis_index("s") * _NC + lax.axis_index("c")
    lane = _vec_iota()
    zero16f = jnp.zeros((_L,), jnp.float32)
    zero16i = jnp.zeros((_L,), jnp.int32)

    def _z(i, c):
        zbuf[pl.ds(i * _L, _L)] = zero16f
        return c
    lax.fori_loop(0, 4096 // _L, _z, 0)

    for r in range(_RPW):
        row = wid * _RPW + r
        base = row * _N

        pltpu.sync_copy(x_hbm.at[pl.ds(base, _N)], kv)
        for cidx in range(8):
            off = 16 + cidx * 4096
            ln = 4096 if cidx < 7 else 4064
            pltpu.sync_copy(zbuf.at[pl.ds(0, ln)],
                            bc_hbm.at[pl.ds(base + off, ln)])

        def _tr(t, c):
            v = kv[pl.ds(t * _L, _L)]
            b = lax.bitcast_convert_type(v, jnp.int32)
            u = jnp.where(v >= 0.0, jnp.invert(jnp.bitwise_or(b, _MSB)), b)
            kv[pl.ds(t * _L, _L)] = lax.bitcast_convert_type(u, jnp.float32)
            return c
        lax.fori_loop(0, _SEG, _tr, 0)

        for p in range(4):
            shift = 8 * p
            src = idx_a if p % 2 == 1 else idx_b
            dst = idx_b if p % 2 == 1 else idx_a

            def _hzero(i, c):
                hist2[pl.ds(i * _L, _L)] = zero16i
                return c
            lax.fori_loop(0, _RADIX, _hzero, 0)

            def _hist(t, c):
                pos = lane * _SEG + t
                if p == 0:
                    cur = pos
                else:
                    cur = plsc.load_gather(src, [pos])
                k = lax.bitcast_convert_type(plsc.load_gather(kv, [cur]), jnp.int32)
                h = _digit(k, shift) * _L + lane
                cnt = plsc.load_gather(hist2, [h])
                plsc.store_scatter(hist2, [h], cnt + 1)
                return c
            lax.fori_loop(0, _SEG, _hist, 0)

            def _scan(i, carry):
                v = hist2[pl.ds(i * _L, _L)]
                inc = plsc.cumsum(v)
                hist2[pl.ds(i * _L, _L)] = inc - v + carry
                return carry + jnp.sum(v)
            lax.fori_loop(0, _RADIX, _scan, jnp.int32(0))

            def _perm(t, c):
                pos = lane * _SEG + t
                if p == 0:
                    cur = pos
                else:
                    cur = plsc.load_gather(src, [pos])
                k = lax.bitcast_convert_type(plsc.load_gather(kv, [cur]), jnp.int32)
                h = _digit(k, shift) * _L + lane
                o = plsc.load_gather(hist2, [h])
                plsc.store_scatter(dst, [o], cur)
                plsc.store_scatter(hist2, [h], o + 1)
                return c
            lax.fori_loop(0, _SEG, _perm, 0)

        pltpu.sync_copy(idx_b, idx_hbm.at[pl.ds(base, _N)])

        def _invert_keys(idx16):
            k = lax.bitcast_convert_type(plsc.load_gather(kv, [idx16]), jnp.int32)
            bits = jnp.where(k < 0, k, jnp.bitwise_and(jnp.invert(k), ~_MSB))
            return lax.bitcast_convert_type(bits, jnp.float32)

        top = _invert_keys(idx_b[pl.ds(0, _L)])
        mh = lane < _G
        mt = jnp.where(mh, top, -3e38)
        eh = jnp.where(mh, jnp.exp(mt - jnp.max(mt)), 0.0)
        headv[...] = eh / jnp.sum(eh)

        bot = _invert_keys(idx_b[pl.ds(_N - _L, _L)])
        tl = 1.0 - bot
        ml = lane >= (_L - _G)
        mtl = jnp.where(ml, tl, -3e38)
        el = jnp.where(ml, jnp.exp(mtl - jnp.max(mtl)), 0.0)
        tailv[...] = -(el / jnp.sum(el))

        pltpu.sync_copy(headv, bc_hbm.at[pl.ds(base, _L)])
        pltpu.sync_copy(tailv, bc_hbm.at[pl.ds(base + _N - _L, _L)])


def kernel(x):
    bc_flat, idx_flat = _sc_sort(x.reshape(-1))
    return (bc_flat.reshape(_B, _N), idx_flat.reshape(_B, _N))


# speedup vs baseline: 1.3035x; 1.1179x over previous
import functools

import jax
import jax.numpy as jnp
from jax import lax
from jax.experimental import pallas as pl
from jax.experimental.pallas import tpu as pltpu
from jax.experimental.pallas import tpu_sc as plsc

_G = 7
_B, _N = 128, 32768
_L = 16
_SEG = _N // _L
_RADIX = 256
_NC, _NS = 2, 16
_NW = _NC * _NS
_RPW = _B // _NW
_MSB = -2147483648


def _vec_iota():
    return lax.iota(jnp.int32, _L)


def _digit(k_i32, shift):
    return jnp.bitwise_and(jnp.right_shift(k_i32, shift), 255)


@functools.partial(
    pl.kernel,
    mesh=plsc.VectorSubcoreMesh(core_axis_name="c", subcore_axis_name="s"),
    compiler_params=pltpu.CompilerParams(needs_layout_passes=False),
    out_type=(
        jax.ShapeDtypeStruct((_B * _N,), jnp.float32),
        jax.ShapeDtypeStruct((_B * _N,), jnp.int32),
    ),
    scratch_types=[
        pltpu.VMEM((_N,), jnp.float32),
        pltpu.VMEM((_N,), jnp.int32),
        pltpu.VMEM((_N,), jnp.int32),
        pltpu.VMEM((_RADIX * _L,), jnp.int32),
        pltpu.VMEM((4096,), jnp.float32),
        pltpu.VMEM((_L,), jnp.float32),
        pltpu.VMEM((_L,), jnp.float32),
    ],
)
def _sc_sort(x_hbm, bc_hbm, idx_hbm, kv, idx_a, idx_b, hist2, zbuf, headv, tailv):
    wid = lax.axis_index("s") * _NC + lax.axis_index("c")
    lane = _vec_iota()
    zero16f = jnp.zeros((_L,), jnp.float32)
    zero16i = jnp.zeros((_L,), jnp.int32)

    def _z(i, c):
        zbuf[pl.ds(i * _L, _L)] = zero16f
        return c
    lax.fori_loop(0, 4096 // _L, _z, 0)

    for r in range(_RPW):
        row = wid * _RPW + r
        base = row * _N

        pltpu.sync_copy(x_hbm.at[pl.ds(base, _N)], kv)
        for cidx in range(8):
            off = 16 + cidx * 4096
            ln = 4096 if cidx < 7 else 4064
            pltpu.sync_copy(zbuf.at[pl.ds(0, ln)],
                            bc_hbm.at[pl.ds(base + off, ln)])

        ones16 = jnp.full((_L,), 1, jnp.int32)
        U = 4

        for p in range(4):
            shift = 8 * p
            src = idx_a if p % 2 == 1 else idx_b
            dst = idx_b if p % 2 == 1 else idx_a

            def _hzero(i, c):
                for u in range(U):
                    hist2[pl.ds((i * U + u) * _L, _L)] = zero16i
                return c
            lax.fori_loop(0, _RADIX // U, _hzero, 0)

            def _hist(tt, c):
                for u in range(U):
                    t = tt * U + u
                    pos = lane * _SEG + t
                    if p == 0:
                        v = plsc.load_gather(kv, [pos])
                        b = lax.bitcast_convert_type(v, jnp.int32)
                        k = jnp.where(v >= 0.0,
                                      jnp.invert(jnp.bitwise_or(b, _MSB)), b)
                        plsc.store_scatter(
                            kv, [pos], lax.bitcast_convert_type(k, jnp.float32))
                    else:
                        cur = plsc.load_gather(src, [pos])
                        k = lax.bitcast_convert_type(
                            plsc.load_gather(kv, [cur]), jnp.int32)
                    h = _digit(k, shift) * _L + lane
                    plsc.addupdate_scatter(hist2, [h], ones16)
                return c
            lax.fori_loop(0, _SEG // U, _hist, 0)

            def _scan(i, carry):
                v = hist2[pl.ds(i * _L, _L)]
                inc = plsc.cumsum(v)
                hist2[pl.ds(i * _L, _L)] = inc - v + carry
                return carry + jnp.sum(v)
            lax.fori_loop(0, _RADIX, _scan, jnp.int32(0))

            def _perm(tt, c):
                for u in range(U):
                    t = tt * U + u
                    pos = lane * _SEG + t
                    if p == 0:
                        cur = pos
                    else:
                        cur = plsc.load_gather(src, [pos])
                    k = lax.bitcast_convert_type(
                        plsc.load_gather(kv, [cur]), jnp.int32)
                    h = _digit(k, shift) * _L + lane
                    o = plsc.load_gather(hist2, [h])
                    plsc.store_scatter(dst, [o], cur)
                    plsc.store_scatter(hist2, [h], o + 1)
                return c
            lax.fori_loop(0, _SEG // U, _perm, 0)

        pltpu.sync_copy(idx_b, idx_hbm.at[pl.ds(base, _N)])

        def _invert_keys(idx16):
            k = lax.bitcast_convert_type(plsc.load_gather(kv, [idx16]), jnp.int32)
            bits = jnp.where(k < 0, k, jnp.bitwise_and(jnp.invert(k), ~_MSB))
            return lax.bitcast_convert_type(bits, jnp.float32)

        top = _invert_keys(idx_b[pl.ds(0, _L)])
        mh = lane < _G
        mt = jnp.where(mh, top, -3e38)
        eh = jnp.where(mh, jnp.exp(mt - jnp.max(mt)), 0.0)
        headv[...] = eh / jnp.sum(eh)

        bot = _invert_keys(idx_b[pl.ds(_N - _L, _L)])
        tl = 1.0 - bot
        ml = lane >= (_L - _G)
        mtl = jnp.where(ml, tl, -3e38)
        el = jnp.where(ml, jnp.exp(mtl - jnp.max(mtl)), 0.0)
        tailv[...] = -(el / jnp.sum(el))

        pltpu.sync_copy(headv, bc_hbm.at[pl.ds(base, _L)])
        pltpu.sync_copy(tailv, bc_hbm.at[pl.ds(base + _N - _L, _L)])


def kernel(x):
    bc_flat, idx_flat = _sc_sort(x.reshape(-1))
    return (bc_flat.reshape(_B, _N), idx_flat.reshape(_B, _N))


# speedup vs baseline: 2.0768x; 1.5932x over previous
import functools

import jax
import jax.numpy as jnp
from jax import lax
from jax.experimental import pallas as pl
from jax.experimental.pallas import tpu as pltpu
from jax.experimental.pallas import tpu_sc as plsc

_G = 7
_B, _N = 128, 32768
_L = 16
_SEG = _N // _L
_RADIX = 2048
_NC, _NS = 2, 16
_NW = _NC * _NS
_RPW = _B // _NW
_MSB = -2147483648


def _vec_iota():
    return lax.iota(jnp.int32, _L)


def _digit(k_i32, shift):
    return jnp.bitwise_and(jnp.right_shift(k_i32, shift), _RADIX - 1)


@functools.partial(
    pl.kernel,
    mesh=plsc.VectorSubcoreMesh(core_axis_name="c", subcore_axis_name="s"),
    compiler_params=pltpu.CompilerParams(needs_layout_passes=False),
    out_type=(
        jax.ShapeDtypeStruct((_B * _N,), jnp.float32),
        jax.ShapeDtypeStruct((_B * _N,), jnp.int32),
    ),
    scratch_types=[
        pltpu.VMEM((_N,), jnp.float32),
        pltpu.VMEM((_N,), jnp.int32),
        pltpu.VMEM((_N,), jnp.int32),
        pltpu.VMEM((_RADIX,), jnp.int32),
        pltpu.VMEM((4096,), jnp.float32),
        pltpu.VMEM((_L,), jnp.float32),
        pltpu.VMEM((_L,), jnp.float32),
    ],
)
def _sc_sort(x_hbm, bc_hbm, idx_hbm, kv, idx_a, idx_b, hist2, zbuf, headv, tailv):
    wid = lax.axis_index("s") * _NC + lax.axis_index("c")
    lane = _vec_iota()
    zero16f = jnp.zeros((_L,), jnp.float32)
    zero16i = jnp.zeros((_L,), jnp.int32)

    def _z(i, c):
        zbuf[pl.ds(i * _L, _L)] = zero16f
        return c
    lax.fori_loop(0, 4096 // _L, _z, 0)

    for r in range(_RPW):
        row = wid * _RPW + r
        base = row * _N

        pltpu.sync_copy(x_hbm.at[pl.ds(base, _N)], kv)
        for cidx in range(8):
            off = 16 + cidx * 4096
            ln = 4096 if cidx < 7 else 4064
            pltpu.sync_copy(zbuf.at[pl.ds(0, ln)],
                            bc_hbm.at[pl.ds(base + off, ln)])

        U = 4

        for p in range(3):
            shift = (0, 11, 21)[p]
            src = idx_b if p == 2 else idx_a
            dst = idx_a if p == 2 else (idx_a if p == 0 else idx_b)

            def _hzero(i, c):
                for u in range(U):
                    hist2[pl.ds((i * U + u) * _L, _L)] = zero16i
                return c
            lax.fori_loop(0, (_RADIX // _L) // U, _hzero, 0)

            def _hist(tt, c):
                for u in range(U):
                    t = tt * U + u
                    if p == 0:
                        v = kv[pl.ds(t * _L, _L)]
                        b = lax.bitcast_convert_type(v, jnp.int32)
                        k = jnp.where(v >= 0.0,
                                      jnp.invert(jnp.bitwise_or(b, _MSB)), b)
                        kv[pl.ds(t * _L, _L)] = \
                            lax.bitcast_convert_type(k, jnp.float32)
                    else:
                        cur = src[pl.ds(t * _L, _L)]
                        k = lax.bitcast_convert_type(
                            plsc.load_gather(kv, [cur]), jnp.int32)
                    d = _digit(k, shift)
                    occ, lastm = plsc.scan_count(d)
                    plsc.addupdate_scatter(hist2, [d], occ, mask=lastm)
                return c
            lax.fori_loop(0, _SEG // U, _hist, 0)

            def _scan(i, carry):
                v = hist2[pl.ds(i * _L, _L)]
                inc = plsc.cumsum(v)
                hist2[pl.ds(i * _L, _L)] = inc - v + carry
                return carry + jnp.sum(v)
            lax.fori_loop(0, _RADIX // _L, _scan, jnp.int32(0))

            def _perm(tt, c):
                for u in range(U):
                    t = tt * U + u
                    if p == 0:
                        cur = t * _L + lane
                    else:
                        cur = src[pl.ds(t * _L, _L)]
                    k = lax.bitcast_convert_type(
                        plsc.load_gather(kv, [cur]), jnp.int32)
                    d = _digit(k, shift)
                    occ, lastm = plsc.scan_count(d)
                    base_o = plsc.load_gather(hist2, [d])
                    plsc.store_scatter(dst, [base_o + occ - 1], cur)
                    plsc.store_scatter(hist2, [d], base_o + occ, mask=lastm)
                return c
            lax.fori_loop(0, _SEG // U, _perm, 0)

        pltpu.sync_copy(idx_a, idx_hbm.at[pl.ds(base, _N)])

        def _invert_keys(idx16):
            k = lax.bitcast_convert_type(plsc.load_gather(kv, [idx16]), jnp.int32)
            bits = jnp.where(k < 0, k, jnp.bitwise_and(jnp.invert(k), ~_MSB))
            return lax.bitcast_convert_type(bits, jnp.float32)

        top = _invert_keys(idx_a[pl.ds(0, _L)])
        mh = lane < _G
        mt = jnp.where(mh, top, -3e38)
        eh = jnp.where(mh, jnp.exp(mt - jnp.max(mt)), 0.0)
        headv[...] = eh / jnp.sum(eh)

        bot = _invert_keys(idx_a[pl.ds(_N - _L, _L)])
        tl = 1.0 - bot
        ml = lane >= (_L - _G)
        mtl = jnp.where(ml, tl, -3e38)
        el = jnp.where(ml, jnp.exp(mtl - jnp.max(mtl)), 0.0)
        tailv[...] = -(el / jnp.sum(el))

        pltpu.sync_copy(headv, bc_hbm.at[pl.ds(base, _L)])
        pltpu.sync_copy(tailv, bc_hbm.at[pl.ds(base + _N - _L, _L)])


def kernel(x):
    bc_flat, idx_flat = _sc_sort(x.reshape(-1))
    return (bc_flat.reshape(_B, _N), idx_flat.reshape(_B, _N))


# speedup vs baseline: 3.4677x; 1.6697x over previous
import functools

import jax
import jax.numpy as jnp
from jax import lax
from jax.experimental import pallas as pl
from jax.experimental.pallas import tpu as pltpu
from jax.experimental.pallas import tpu_sc as plsc

_G = 7
_B, _N = 128, 32768
_L = 16
_SEG = _N // _L
_RADIX = 2048
_NC, _NS = 2, 16
_NW = _NC * _NS
_RPW = _B // _NW
_MSB = -2147483648


def _vec_iota():
    return lax.iota(jnp.int32, _L)


def _digit(k_i32, shift):
    return jnp.bitwise_and(jnp.right_shift(k_i32, shift), _RADIX - 1)


@functools.partial(
    pl.kernel,
    mesh=plsc.VectorSubcoreMesh(core_axis_name="c", subcore_axis_name="s"),
    compiler_params=pltpu.CompilerParams(needs_layout_passes=False),
    out_type=(
        jax.ShapeDtypeStruct((_B * _N,), jnp.float32),
        jax.ShapeDtypeStruct((_B * _N,), jnp.int32),
    ),
    scratch_types=[
        pltpu.VMEM((_N,), jnp.float32),
        pltpu.VMEM((_N,), jnp.int32),
        pltpu.VMEM((_N,), jnp.int32),
        pltpu.VMEM((_RADIX,), jnp.int32),
        pltpu.VMEM((4096,), jnp.float32),
        pltpu.VMEM((_L,), jnp.float32),
        pltpu.VMEM((_L,), jnp.float32),
    ],
)
def _sc_sort(x_hbm, bc_hbm, idx_hbm, kv, idx_a, idx_b, hist2, zbuf, headv, tailv):
    wid = lax.axis_index("s") * _NC + lax.axis_index("c")
    lane = _vec_iota()
    zero16f = jnp.zeros((_L,), jnp.float32)
    zero16i = jnp.zeros((_L,), jnp.int32)

    def _z(i, c):
        zbuf[pl.ds(i * _L, _L)] = zero16f
        return c
    lax.fori_loop(0, 4096 // _L, _z, 0)

    for r in range(_RPW):
        row = wid * _RPW + r
        base = row * _N

        pltpu.sync_copy(x_hbm.at[pl.ds(base, _N)], kv)
        for cidx in range(8):
            off = 16 + cidx * 4096
            ln = 4096 if cidx < 7 else 4064
            pltpu.sync_copy(zbuf.at[pl.ds(0, ln)],
                            bc_hbm.at[pl.ds(base + off, ln)])

        U = 8

        for p in range(3):
            shift = (0, 11, 21)[p]
            src = idx_b if p == 2 else idx_a
            dst = idx_a if p == 2 else (idx_a if p == 0 else idx_b)

            def _hzero(i, c):
                for u in range(U):
                    hist2[pl.ds((i * U + u) * _L, _L)] = zero16i
                return c
            lax.fori_loop(0, (_RADIX // _L) // U, _hzero, 0)

            @plsc.parallel_loop(0, _SEG, unroll=8)
            def _hist(t):
                if p == 0:
                    v = kv[pl.ds(t * _L, _L)]
                    b = lax.bitcast_convert_type(v, jnp.int32)
                    k = jnp.where(v >= 0.0,
                                  jnp.invert(jnp.bitwise_or(b, _MSB)), b)
                    kv[pl.ds(t * _L, _L)] = \
                        lax.bitcast_convert_type(k, jnp.float32)
                else:
                    cur = src[pl.ds(t * _L, _L)]
                    k = lax.bitcast_convert_type(
                        plsc.load_gather(kv, [cur]), jnp.int32)
                d = _digit(k, shift)
                occ, lastm = plsc.scan_count(d)
                plsc.addupdate_scatter(hist2, [d], occ, mask=lastm)

            def _scan(i, carry):
                v = hist2[pl.ds(i * _L, _L)]
                inc = plsc.cumsum(v)
                hist2[pl.ds(i * _L, _L)] = inc - v + carry
                return carry + jnp.sum(v)
            lax.fori_loop(0, _RADIX // _L, _scan, jnp.int32(0))

            def _perm(tt, c):
                for u in range(U):
                    t = tt * U + u
                    if p == 0:
                        cur = t * _L + lane
                    else:
                        cur = src[pl.ds(t * _L, _L)]
                    k = lax.bitcast_convert_type(
                        plsc.load_gather(kv, [cur]), jnp.int32)
                    d = _digit(k, shift)
                    occ, lastm = plsc.scan_count(d)
                    base_o = plsc.load_gather(hist2, [d])
                    plsc.store_scatter(dst, [base_o + occ - 1], cur)
                    plsc.store_scatter(hist2, [d], base_o + occ, mask=lastm)
                return c
            lax.fori_loop(0, _SEG // U, _perm, 0)

        pltpu.sync_copy(idx_a, idx_hbm.at[pl.ds(base, _N)])

        def _invert_keys(idx16):
            k = lax.bitcast_convert_type(plsc.load_gather(kv, [idx16]), jnp.int32)
            bits = jnp.where(k < 0, k, jnp.bitwise_and(jnp.invert(k), ~_MSB))
            return lax.bitcast_convert_type(bits, jnp.float32)

        top = _invert_keys(idx_a[pl.ds(0, _L)])
        mh = lane < _G
        mt = jnp.where(mh, top, -3e38)
        eh = jnp.where(mh, jnp.exp(mt - jnp.max(mt)), 0.0)
        headv[...] = eh / jnp.sum(eh)

        bot = _invert_keys(idx_a[pl.ds(_N - _L, _L)])
        tl = 1.0 - bot
        ml = lane >= (_L - _G)
        mtl = jnp.where(ml, tl, -3e38)
        el = jnp.where(ml, jnp.exp(mtl - jnp.max(mtl)), 0.0)
        tailv[...] = -(el / jnp.sum(el))

        pltpu.sync_copy(headv, bc_hbm.at[pl.ds(base, _L)])
        pltpu.sync_copy(tailv, bc_hbm.at[pl.ds(base + _N - _L, _L)])


def kernel(x):
    bc_flat, idx_flat = _sc_sort(x.reshape(-1))
    return (bc_flat.reshape(_B, _N), idx_flat.reshape(_B, _N))
